# baseline (speedup 1.0000x reference)
import functools

import jax
import jax.numpy as jnp
from jax import lax
from jax.experimental import pallas as pl
from jax.experimental.pallas import tpu as pltpu
from jax.experimental.pallas import tpu_sc as plsc

_L = 128
_CHUNK = 512


def _make_gather(B, D, n_workers):
    b_per_w = B // n_workers
    n_chunks = b_per_w // _CHUNK
    mesh = plsc.VectorSubcoreMesh(core_axis_name="c", subcore_axis_name="s")
    nc = mesh.num_cores

    @functools.partial(
        pl.kernel,
        out_type=jax.ShapeDtypeStruct((B, D), jnp.float32),
        mesh=mesh,
        scratch_types=[
            pltpu.VMEM((_CHUNK // _L, _L), jnp.int32),
            pltpu.VMEM((_CHUNK, D), jnp.float32),
            pltpu.SemaphoreType.DMA,
        ],
        compiler_params=pltpu.CompilerParams(use_tc_tiling_on_sc=False),
    )
    def k(idx_hbm, table_hbm, out_hbm, idx_v, rows_v, sem):
        wid = lax.axis_index("s") * nc + lax.axis_index("c")
        base = wid * b_per_w

        def body(g, carry):
            row0 = base + g * _CHUNK
            pltpu.sync_copy(idx_hbm.at[wid * n_chunks + g], idx_v)
            copies = [
                pltpu.async_copy(
                    table_hbm.at[idx_v.at[j]],
                    rows_v.at[pl.ds(j * _L, _L), :],
                    sem,
                )
                for j in range(_CHUNK // _L)
            ]
            for c in copies:
                c.wait()
            pltpu.sync_copy(rows_v, out_hbm.at[pl.ds(row0, _CHUNK), :])
            return carry

        lax.fori_loop(0, n_chunks, body, 0)

    return k


def kernel(tokens, table):
    b0, s = tokens.shape
    _, d = table.shape
    idx = tokens.reshape(-1).astype(jnp.int32)
    b = idx.shape[0]
    idx2 = idx.reshape(b // _CHUNK, _CHUNK // _L, _L)
    out = _make_gather(b, d, 32)(idx2, table)
    return out.reshape(b0, s, d)


# speedup vs baseline: 1.0468x; 1.0468x over previous
import functools

import jax
import jax.numpy as jnp
from jax import lax
from jax.experimental import pallas as pl
from jax.experimental.pallas import tpu as pltpu
from jax.experimental.pallas import tpu_sc as plsc

_L = 128
_CHUNK = 512
_NBUF = 2


def _make_gather(B, D, n_workers):
    b_per_w = B // n_workers
    n_chunks = b_per_w // _CHUNK
    assert n_chunks % _NBUF == 0
    mesh = plsc.VectorSubcoreMesh(core_axis_name="c", subcore_axis_name="s")
    nc = mesh.num_cores

    @functools.partial(
        pl.kernel,
        out_type=jax.ShapeDtypeStruct((B, D), jnp.float32),
        mesh=mesh,
        scratch_types=[
            pltpu.VMEM((n_chunks, _CHUNK // _L, _L), jnp.int32),
            pltpu.VMEM((_NBUF, _CHUNK, D), jnp.float32),
            pltpu.SemaphoreType.DMA,
            pltpu.SemaphoreType.DMA,
        ],
        compiler_params=pltpu.CompilerParams(use_tc_tiling_on_sc=False),
    )
    def k(idx_hbm, table_hbm, out_hbm, idx_v, rows_v, sem0, sem1):
        wid = lax.axis_index("s") * nc + lax.axis_index("c")
        c0 = wid * n_chunks
        sems = (sem0, sem1)

        pltpu.sync_copy(idx_hbm.at[pl.ds(c0, n_chunks)], idx_v)

        def fire(g, slot):
            for j in range(_CHUNK // _L):
                pltpu.async_copy(
                    table_hbm.at[idx_v.at[g, j]],
                    rows_v.at[slot, pl.ds(j * _L, _L), :],
                    sems[slot],
                )

        def drain(g, slot):
            for j in range(_CHUNK // _L):
                pltpu.make_async_copy(
                    table_hbm.at[idx_v.at[g, j]],
                    rows_v.at[slot, pl.ds(j * _L, _L), :],
                    sems[slot],
                ).wait()

        for b in range(_NBUF):
            fire(b, b)

        def outer(i, carry):
            for s in range(_NBUF):
                g = i * _NBUF + s
                drain(g, s)
                pltpu.sync_copy(
                    rows_v.at[s],
                    out_hbm.at[pl.ds((c0 + g) * _CHUNK, _CHUNK), :],
                )

                @pl.when(g + _NBUF < n_chunks)
                def _():
                    fire(g + _NBUF, s)

            return carry

        lax.fori_loop(0, n_chunks // _NBUF, outer, 0)

    return k


def kernel(tokens, table):
    b0, s = tokens.shape
    _, d = table.shape
    idx = tokens.reshape(-1).astype(jnp.int32)
    b = idx.shape[0]
    idx2 = idx.reshape(b // _CHUNK, _CHUNK // _L, _L)
    out = _make_gather(b, d, 32)(idx2, table)
    return out.reshape(b0, s, d)
